# baseline (speedup 1.0000x reference)
import jax
import jax.numpy as jnp
from jax.experimental import pallas as pl
from jax.experimental.pallas import tpu as pltpu

H_BLK = 1536
RING = 4
LOOK = 3


def _moe_kernel(x_ref, wg_ref, wn_ref, eps_ref, w1_ref, b1_ref, b2_ref,
                w2_hbm, out_ref, acc_ref, w_ref, buf2_ref, sem2):
    e = pl.program_id(0)
    c = pl.program_id(1)
    n_chunk = pl.num_programs(1)
    n_exp = wg_ref.shape[1]
    n_chunks_total = w2_hbm.shape[0]
    i = e * n_chunk + c

    def cp2(idx, slot):
        return pltpu.make_async_copy(w2_hbm.at[pl.ds(idx, 1)],
                                     buf2_ref.at[pl.ds(slot, 1)],
                                     sem2.at[slot])

    @pl.when((e == 0) & (c == 0))
    def _gating():
        for k in range(LOOK + 1):
            cp2(k, k).start()
        xv = x_ref[...]
        g = jnp.dot(xv, wg_ref[...], preferred_element_type=jnp.float32)
        n = jnp.dot(xv, wn_ref[...], preferred_element_type=jnp.float32)
        logits = g + jax.nn.softplus(n) * eps_ref[...]
        lane = jax.lax.broadcasted_iota(jnp.int32, logits.shape, 1)
        i1 = jnp.argmax(logits, axis=1)[:, None]
        v1 = jnp.max(logits, axis=1)[:, None]
        oh1 = lane == i1
        masked = jnp.where(oh1, -jnp.inf, logits)
        i2 = jnp.argmax(masked, axis=1)[:, None]
        v2 = jnp.max(masked, axis=1)[:, None]
        oh2 = lane == i2
        e2 = jnp.exp(v2 - v1)
        denom = 1.0 + e2
        w_ref[...] = jnp.where(oh1, 1.0 / denom,
                               jnp.where(oh2, e2 / denom, 0.0))
        out_ref[...] = jnp.zeros_like(out_ref)

    @pl.when(c == 0)
    def _init_acc():
        acc_ref[...] = jnp.zeros_like(acc_ref)

    slot = jax.lax.rem(i, RING)
    cp2(i, slot).wait()

    h = jnp.dot(x_ref[...], w1_ref[0], preferred_element_type=jnp.float32)
    h = jnp.maximum(h + b1_ref[0], 0.0)
    for s in range(RING):
        @pl.when(slot == s)
        def _consume(s=s):
            acc_ref[...] += jnp.dot(h, buf2_ref[s],
                                    preferred_element_type=jnp.float32)

    @pl.when(i + LOOK + 1 < n_chunks_total)
    def _refill():
        cp2(i + LOOK + 1, jax.lax.rem(i + LOOK + 1, RING)).start()

    @pl.when(c == n_chunk - 1)
    def _combine():
        lane = jax.lax.broadcasted_iota(jnp.int32, (out_ref.shape[0], n_exp), 1)
        we = jnp.sum(jnp.where(lane == e, w_ref[...], 0.0), axis=1,
                     keepdims=True)
        out_ref[...] += we * (acc_ref[...] + b2_ref[0])


def kernel(x, Wg, Wnoise, W1, b1, W2, b2):
    b, c, d = x.shape
    n_exp, _, d_hid = W1.shape
    t = b * c
    x2 = x.reshape(t, d)
    eps = jax.random.normal(jax.random.key(42), (b, c, n_exp),
                            dtype=x.dtype).reshape(t, n_exp)
    n_chunk = d_hid // H_BLK
    w2r = W2.reshape(n_exp * n_chunk, H_BLK, d)
    out = pl.pallas_call(
        _moe_kernel,
        grid=(n_exp, n_chunk),
        in_specs=[
            pl.BlockSpec((t, d), lambda e, c: (0, 0)),
            pl.BlockSpec((d, n_exp), lambda e, c: (0, 0)),
            pl.BlockSpec((d, n_exp), lambda e, c: (0, 0)),
            pl.BlockSpec((t, n_exp), lambda e, c: (0, 0)),
            pl.BlockSpec((1, d, H_BLK), lambda e, c: (e, 0, c)),
            pl.BlockSpec((1, 1, H_BLK), lambda e, c: (e, 0, c)),
            pl.BlockSpec((1, 1, d), lambda e, c: (e, 0, 0)),
            pl.BlockSpec(memory_space=pltpu.MemorySpace.HBM),
        ],
        out_specs=pl.BlockSpec((t, d), lambda e, c: (0, 0)),
        out_shape=jax.ShapeDtypeStruct((t, d), x.dtype),
        scratch_shapes=[
            pltpu.VMEM((t, d), jnp.float32),
            pltpu.VMEM((t, n_exp), jnp.float32),
            pltpu.VMEM((RING, H_BLK, d), jnp.float32),
            pltpu.SemaphoreType.DMA((RING,)),
        ],
        compiler_params=pltpu.CompilerParams(
            dimension_semantics=("arbitrary", "arbitrary")),
    )(x2, Wg.T, Wnoise.T, eps, W1, b1[:, None, :], b2[:, None, :], w2r)
    return out.reshape(b, c, d)


# speedup vs baseline: 1.0146x; 1.0146x over previous
import jax
import jax.numpy as jnp
from jax.experimental import pallas as pl
from jax.experimental.pallas import tpu as pltpu

H_BLK = 1536
RING = 4
LOOK = 2


def _moe_kernel(x_ref, wg_ref, wn_ref, eps_ref, w1_ref, b1_ref, b2_ref,
                w2_hbm, out_ref, acc_ref, w_ref, buf2_ref, sem2):
    e = pl.program_id(0)
    c = pl.program_id(1)
    n_chunk = pl.num_programs(1)
    n_exp = wg_ref.shape[1]
    n_chunks_total = w2_hbm.shape[0]
    i = e * n_chunk + c

    def cp2(idx, slot):
        return pltpu.make_async_copy(w2_hbm.at[pl.ds(idx, 1)],
                                     buf2_ref.at[pl.ds(slot, 1)],
                                     sem2.at[slot])

    @pl.when((e == 0) & (c == 0))
    def _gating():
        for k in range(LOOK + 1):
            cp2(k, k).start()
        xv = x_ref[...]
        g = jnp.dot(xv, wg_ref[...], preferred_element_type=jnp.float32)
        n = jnp.dot(xv, wn_ref[...], preferred_element_type=jnp.float32)
        logits = g + jax.nn.softplus(n) * eps_ref[...]
        lane = jax.lax.broadcasted_iota(jnp.int32, logits.shape, 1)
        i1 = jnp.argmax(logits, axis=1)[:, None]
        v1 = jnp.max(logits, axis=1)[:, None]
        oh1 = lane == i1
        masked = jnp.where(oh1, -jnp.inf, logits)
        i2 = jnp.argmax(masked, axis=1)[:, None]
        v2 = jnp.max(masked, axis=1)[:, None]
        oh2 = lane == i2
        e2 = jnp.exp(v2 - v1)
        denom = 1.0 + e2
        w_ref[...] = jnp.where(oh1, 1.0 / denom,
                               jnp.where(oh2, e2 / denom, 0.0))
        out_ref[...] = jnp.zeros_like(out_ref)

    @pl.when(c == 0)
    def _init_acc():
        acc_ref[...] = jnp.zeros_like(acc_ref)

    slot = jax.lax.rem(i, RING)
    cp2(i, slot).wait()

    h = jnp.dot(x_ref[...], w1_ref[0], preferred_element_type=jnp.float32)
    h = jnp.maximum(h + b1_ref[0], 0.0)
    for s in range(RING):
        @pl.when(slot == s)
        def _consume(s=s):
            acc_ref[...] += jnp.dot(h, buf2_ref[s],
                                    preferred_element_type=jnp.float32)

    @pl.when(i + LOOK + 1 < n_chunks_total)
    def _refill():
        cp2(i + LOOK + 1, jax.lax.rem(i + LOOK + 1, RING)).start()

    @pl.when(c == n_chunk - 1)
    def _combine():
        lane = jax.lax.broadcasted_iota(jnp.int32, (out_ref.shape[0], n_exp), 1)
        we = jnp.sum(jnp.where(lane == e, w_ref[...], 0.0), axis=1,
                     keepdims=True)
        out_ref[...] += we * (acc_ref[...] + b2_ref[0])


def kernel(x, Wg, Wnoise, W1, b1, W2, b2):
    b, c, d = x.shape
    n_exp, _, d_hid = W1.shape
    t = b * c
    x2 = x.reshape(t, d)
    eps = jax.random.normal(jax.random.key(42), (b, c, n_exp),
                            dtype=x.dtype).reshape(t, n_exp)
    n_chunk = d_hid // H_BLK
    w2r = W2.reshape(n_exp * n_chunk, H_BLK, d)
    out = pl.pallas_call(
        _moe_kernel,
        grid=(n_exp, n_chunk),
        in_specs=[
            pl.BlockSpec((t, d), lambda e, c: (0, 0)),
            pl.BlockSpec((d, n_exp), lambda e, c: (0, 0)),
            pl.BlockSpec((d, n_exp), lambda e, c: (0, 0)),
            pl.BlockSpec((t, n_exp), lambda e, c: (0, 0)),
            pl.BlockSpec((1, d, H_BLK), lambda e, c: (e, 0, c)),
            pl.BlockSpec((1, 1, H_BLK), lambda e, c: (e, 0, c)),
            pl.BlockSpec((1, 1, d), lambda e, c: (e, 0, 0)),
            pl.BlockSpec(memory_space=pltpu.MemorySpace.HBM),
        ],
        out_specs=pl.BlockSpec((t, d), lambda e, c: (0, 0)),
        out_shape=jax.ShapeDtypeStruct((t, d), x.dtype),
        scratch_shapes=[
            pltpu.VMEM((t, d), jnp.float32),
            pltpu.VMEM((t, n_exp), jnp.float32),
            pltpu.VMEM((RING, H_BLK, d), jnp.float32),
            pltpu.SemaphoreType.DMA((RING,)),
        ],
        compiler_params=pltpu.CompilerParams(
            dimension_semantics=("arbitrary", "arbitrary")),
    )(x2, Wg.T, Wnoise.T, eps, W1, b1[:, None, :], b2[:, None, :], w2r)
    return out.reshape(b, c, d)
